# baseline (speedup 1.0000x reference)
import functools

import jax
import jax.numpy as jnp
from jax.experimental import pallas as pl
from jax.experimental.pallas import tpu as pltpu

_ITERS = 14
_SPLIT = 14


def _split_reduce(arr, op, combine, n):
    step = n // _SPLIT
    parts = [
        op(arr[:, j * step:(j + 1) * step], axis=1, keepdims=True)
        for j in range(_SPLIT)
    ]
    while len(parts) > 1:
        nxt = [combine(parts[i], parts[i + 1])
               for i in range(0, len(parts) - 1, 2)]
        if len(parts) % 2:
            nxt.append(parts[-1])
        parts = nxt
    return parts[0]


def _pool_body(t_ref, x_ref, o_ref, *, k, n, iters):
    xb = x_ref[...]
    maxv = _split_reduce(xb, jnp.max, jnp.maximum, n)
    minv = _split_reduce(xb, jnp.min, jnp.minimum, n)
    kf = jnp.float32(k)

    def step(_, carry):
        lo, hi = carry
        mid = 0.5 * (lo + hi)
        cnt = _split_reduce(jnp.where(xb >= mid, 1.0, 0.0), jnp.sum, jnp.add, n)
        ok = cnt >= kf
        return jnp.where(ok, mid, lo), jnp.where(ok, hi, mid)

    lo, _ = jax.lax.fori_loop(0, iters, step, (minv, maxv))
    t = lo
    ge = xb >= t
    cnt_ge = _split_reduce(jnp.where(ge, 1.0, 0.0), jnp.sum, jnp.add, n)
    sum_ge = _split_reduce(jnp.where(ge, xb, 0.0), jnp.sum, jnp.add, n)
    topk_sum = sum_ge - (cnt_ge - kf) * t
    avg = topk_sum / kf

    denom = maxv + 1e-6
    s = jnp.minimum(t / denom, maxv / denom)
    ts = jax.nn.sigmoid(t_ref[0, 0])
    logits = (s - ts) / 0.1
    gate_soft = jax.nn.sigmoid(logits)
    gate_hard = (logits >= 0).astype(jnp.float32)
    gate = (gate_hard - gate_soft) + gate_soft
    o_ref[...] = gate * maxv + (1.0 - gate) * avg


def _pool_rows(xf, t2, *, k, n, r_blk, iters):
    rows_c = xf.shape[0]
    return pl.pallas_call(
        functools.partial(_pool_body, k=k, n=n, iters=iters),
        grid=(rows_c // r_blk,),
        in_specs=[
            pl.BlockSpec(memory_space=pltpu.SMEM),
            pl.BlockSpec((r_blk, n), lambda i: (i, 0)),
        ],
        out_specs=pl.BlockSpec((r_blk, 1), lambda i: (i, 0)),
        out_shape=jax.ShapeDtypeStruct((rows_c, 1), jnp.float32),
        compiler_params=pltpu.CompilerParams(
            dimension_semantics=("arbitrary",),
        ),
    )(t2, xf)


def kernel(x, T):
    B, C, H, W = x.shape
    n = H * W
    k = max(1, int(n * 0.1))
    r_blk = 32
    chunks = 4
    assert B % chunks == 0 and (B // chunks) * C % r_blk == 0
    t2 = jnp.reshape(T, (1, 1)).astype(jnp.float32)
    bc = B // chunks
    outs = []
    for i in range(chunks):
        xf = x[i * bc:(i + 1) * bc].reshape(bc * C, n)
        outs.append(_pool_rows(xf, t2, k=k, n=n, r_blk=r_blk, iters=_ITERS))
    return jnp.concatenate(outs, axis=0).reshape(B, C)


# speedup vs baseline: 1.2879x; 1.2879x over previous
import functools

import jax
import jax.numpy as jnp
from jax.experimental import pallas as pl
from jax.experimental.pallas import tpu as pltpu

_ITERS = 14


def _reduce(arr, op):
    return op(arr, axis=(1, 2), keepdims=True)


def _pool_body(t_ref, x_ref, o_ref, *, k, iters):
    xb = x_ref[0]
    maxv = _reduce(xb, jnp.max)
    minv = _reduce(xb, jnp.min)
    kf = jnp.float32(k)

    def step(_, carry):
        lo, hi = carry
        mid = 0.5 * (lo + hi)
        cnt = _reduce(jnp.where(xb >= mid, 1.0, 0.0), jnp.sum)
        ok = cnt >= kf
        return jnp.where(ok, mid, lo), jnp.where(ok, hi, mid)

    lo, _ = jax.lax.fori_loop(0, iters, step, (minv, maxv))
    t = lo
    ge = xb >= t
    cnt_ge = _reduce(jnp.where(ge, 1.0, 0.0), jnp.sum)
    sum_ge = _reduce(jnp.where(ge, xb, 0.0), jnp.sum)
    topk_sum = sum_ge - (cnt_ge - kf) * t
    avg = topk_sum / kf

    denom = maxv + 1e-6
    s = jnp.minimum(t / denom, maxv / denom)
    ts = jax.nn.sigmoid(t_ref[0, 0])
    logits = (s - ts) / 0.1
    gate_soft = jax.nn.sigmoid(logits)
    gate_hard = (logits >= 0).astype(jnp.float32)
    gate = (gate_hard - gate_soft) + gate_soft
    pooled = gate * maxv + (1.0 - gate) * avg
    o_ref[...] = pooled[:, :, 0].reshape(1, 1, -1)


def kernel(x, T):
    B, C, H, W = x.shape
    n = H * W
    k = max(1, int(n * 0.1))
    rows = B * C
    c_blk = 32
    assert C % c_blk == 0
    t2 = jnp.reshape(T, (1, 1)).astype(jnp.float32)

    out = pl.pallas_call(
        functools.partial(_pool_body, k=k, iters=_ITERS),
        grid=(B, C // c_blk),
        in_specs=[
            pl.BlockSpec(memory_space=pltpu.SMEM),
            pl.BlockSpec((1, c_blk, H, W), lambda b, c: (b, c, 0, 0)),
        ],
        out_specs=pl.BlockSpec((1, 1, c_blk),
                               lambda b, c: (b * (C // c_blk) + c, 0, 0)),
        out_shape=jax.ShapeDtypeStruct((rows // c_blk, 1, c_blk), jnp.float32),
        compiler_params=pltpu.CompilerParams(
            dimension_semantics=("arbitrary", "arbitrary"),
        ),
    )(t2, x)
    return out.reshape(B, C)


# speedup vs baseline: 1.4050x; 1.0910x over previous
import functools

import jax
import jax.numpy as jnp
from jax.experimental import pallas as pl
from jax.experimental.pallas import tpu as pltpu

_ITERS = 12


def _reduce(arr, op):
    return op(arr, axis=(1, 2), keepdims=True)


def _pool_body(t_ref, x_ref, o_ref, *, k, iters):
    xb = x_ref[0]
    maxv = _reduce(xb, jnp.max)
    minv = _reduce(xb, jnp.min)
    kf = jnp.float32(k)

    def step(_, carry):
        lo, hi = carry
        mid = 0.5 * (lo + hi)
        cnt = _reduce(jnp.where(xb >= mid, 1.0, 0.0), jnp.sum)
        ok = cnt >= kf
        return jnp.where(ok, mid, lo), jnp.where(ok, hi, mid)

    lo, _ = jax.lax.fori_loop(0, iters, step, (minv, maxv))
    t = lo
    ge = xb >= t
    cnt_ge = _reduce(jnp.where(ge, 1.0, 0.0), jnp.sum)
    sum_ge = _reduce(jnp.where(ge, xb, 0.0), jnp.sum)
    topk_sum = sum_ge - (cnt_ge - kf) * t
    avg = topk_sum / kf

    denom = maxv + 1e-6
    s = jnp.minimum(t / denom, maxv / denom)
    ts = jax.nn.sigmoid(t_ref[0, 0])
    logits = (s - ts) / 0.1
    gate_soft = jax.nn.sigmoid(logits)
    gate_hard = (logits >= 0).astype(jnp.float32)
    gate = (gate_hard - gate_soft) + gate_soft
    pooled = gate * maxv + (1.0 - gate) * avg
    o_ref[...] = pooled[:, :, 0].reshape(1, 1, -1)


def kernel(x, T):
    B, C, H, W = x.shape
    n = H * W
    k = max(1, int(n * 0.1))
    rows = B * C
    c_blk = 32
    assert C % c_blk == 0
    t2 = jnp.reshape(T, (1, 1)).astype(jnp.float32)

    out = pl.pallas_call(
        functools.partial(_pool_body, k=k, iters=_ITERS),
        grid=(B, C // c_blk),
        in_specs=[
            pl.BlockSpec(memory_space=pltpu.SMEM),
            pl.BlockSpec((1, c_blk, H, W), lambda b, c: (b, c, 0, 0)),
        ],
        out_specs=pl.BlockSpec((1, 1, c_blk),
                               lambda b, c: (b * (C // c_blk) + c, 0, 0)),
        out_shape=jax.ShapeDtypeStruct((rows // c_blk, 1, c_blk), jnp.float32),
        compiler_params=pltpu.CompilerParams(
            dimension_semantics=("arbitrary", "arbitrary"),
        ),
    )(t2, x)
    return out.reshape(B, C)


# speedup vs baseline: 1.4353x; 1.0215x over previous
import functools

import jax
import jax.numpy as jnp
from jax.experimental import pallas as pl
from jax.experimental.pallas import tpu as pltpu

_ITERS = 12


def _reduce(arr, op):
    return op(arr, axis=(1, 2), keepdims=True)


def _pool_body(t_ref, x_ref, o_ref, *, k, iters):
    xb = x_ref[0]
    maxv = _reduce(xb, jnp.max)
    minv = _reduce(xb, jnp.min)
    kf = jnp.float32(k)

    def step(_, carry):
        lo, hi = carry
        mid = 0.5 * (lo + hi)
        cnt = _reduce(jnp.where(xb >= mid, 1.0, 0.0), jnp.sum)
        ok = cnt >= kf
        return jnp.where(ok, mid, lo), jnp.where(ok, hi, mid)

    lo, _ = jax.lax.fori_loop(0, iters, step, (minv, maxv))
    t = lo
    ge = xb >= t
    cnt_ge = _reduce(jnp.where(ge, 1.0, 0.0), jnp.sum)
    sum_ge = _reduce(jnp.where(ge, xb, 0.0), jnp.sum)
    topk_sum = sum_ge - (cnt_ge - kf) * t
    avg = topk_sum / kf

    denom = maxv + 1e-6
    s = jnp.minimum(t / denom, maxv / denom)
    ts = jax.nn.sigmoid(t_ref[0, 0])
    logits = (s - ts) / 0.1
    gate_soft = jax.nn.sigmoid(logits)
    gate_hard = (logits >= 0).astype(jnp.float32)
    gate = (gate_hard - gate_soft) + gate_soft
    pooled = gate * maxv + (1.0 - gate) * avg
    o_ref[...] = pooled[:, :, 0].reshape(1, 1, -1)


def kernel(x, T):
    B, C, H, W = x.shape
    n = H * W
    k = max(1, int(n * 0.1))
    rows = B * C
    c_blk = 64
    assert C % c_blk == 0
    t2 = jnp.reshape(T, (1, 1)).astype(jnp.float32)

    out = pl.pallas_call(
        functools.partial(_pool_body, k=k, iters=_ITERS),
        grid=(B, C // c_blk),
        in_specs=[
            pl.BlockSpec(memory_space=pltpu.SMEM),
            pl.BlockSpec((1, c_blk, H, W), lambda b, c: (b, c, 0, 0)),
        ],
        out_specs=pl.BlockSpec((1, 1, c_blk),
                               lambda b, c: (b * (C // c_blk) + c, 0, 0)),
        out_shape=jax.ShapeDtypeStruct((rows // c_blk, 1, c_blk), jnp.float32),
        compiler_params=pltpu.CompilerParams(
            dimension_semantics=("arbitrary", "arbitrary"),
        ),
    )(t2, x)
    return out.reshape(B, C)


# speedup vs baseline: 1.6025x; 1.1165x over previous
import functools

import jax
import jax.numpy as jnp
from jax import lax
from jax.experimental import pallas as pl
from jax.experimental.pallas import tpu as pltpu
from jax.experimental.pallas import tpu_sc as plsc

_ITERS = 12
_TC_CBLK = 64
_SC_C = 64


def _reduce(arr, op):
    return op(arr, axis=(1, 2), keepdims=True)


def _pool_body(t_ref, x_ref, o_ref, *, k, iters):
    xb = x_ref[0]
    maxv = _reduce(xb, jnp.max)
    minv = _reduce(xb, jnp.min)
    kf = jnp.float32(k)

    def step(_, carry):
        lo, hi = carry
        mid = 0.5 * (lo + hi)
        cnt = _reduce(jnp.where(xb >= mid, 1.0, 0.0), jnp.sum)
        ok = cnt >= kf
        return jnp.where(ok, mid, lo), jnp.where(ok, hi, mid)

    lo, _ = lax.fori_loop(0, iters, step, (minv, maxv))
    t = lo
    ge = xb >= t
    cnt_ge = _reduce(jnp.where(ge, 1.0, 0.0), jnp.sum)
    sum_ge = _reduce(jnp.where(ge, xb, 0.0), jnp.sum)
    topk_sum = sum_ge - (cnt_ge - kf) * t
    avg = topk_sum / kf

    denom = maxv + 1e-6
    s = jnp.minimum(t / denom, maxv / denom)
    ts = jax.nn.sigmoid(t_ref[0, 0])
    logits = (s - ts) / 0.1
    gate_soft = jax.nn.sigmoid(logits)
    gate_hard = (logits >= 0).astype(jnp.float32)
    gate = (gate_hard - gate_soft) + gate_soft
    pooled = gate * maxv + (1.0 - gate) * avg
    o_ref[...] = pooled[:, :, 0].reshape(1, 1, -1)


def _tc_pool(x, t2, *, k, tc_c):
    B, C, H, W = x.shape
    c_blk = _TC_CBLK
    nc = tc_c // c_blk
    out = pl.pallas_call(
        functools.partial(_pool_body, k=k, iters=_ITERS),
        grid=(B, nc),
        in_specs=[
            pl.BlockSpec(memory_space=pltpu.SMEM),
            pl.BlockSpec((1, c_blk, H, W), lambda b, c: (b, c, 0, 0)),
        ],
        out_specs=pl.BlockSpec((1, 1, c_blk), lambda b, c: (b * nc + c, 0, 0)),
        out_shape=jax.ShapeDtypeStruct((B * nc, 1, c_blk), jnp.float32),
        compiler_params=pltpu.CompilerParams(
            dimension_semantics=("arbitrary", "arbitrary"),
        ),
    )(t2, x)
    return out.reshape(B, tc_c)


def _sc_pool(x, tvec, *, k, c0):
    B, C, H, W = x.shape
    kf = jnp.float32(k)
    n_sub = _SC_C // 16
    mesh = plsc.VectorSubcoreMesh(core_axis_name="c", subcore_axis_name="s")

    @functools.partial(
        pl.kernel,
        mesh=mesh,
        out_type=jax.ShapeDtypeStruct((B, _SC_C), jnp.float32),
        scratch_types=[
            pltpu.VMEM((H, W), jnp.float32),
            pltpu.VMEM((16,), jnp.float32),
            pltpu.VMEM((16,), jnp.float32),
        ],
    )
    def body(x_hbm, t_hbm, out_hbm, img, tv, pv):
        wid = lax.axis_index("s") * 2 + lax.axis_index("c")
        b = wid // n_sub
        cg = wid % n_sub
        pltpu.sync_copy(t_hbm, tv)
        tvv = tv[...]
        ts_vec = 1.0 / (1.0 + jnp.exp(-tvv))

        def lane_fold(vec, op):
            s = vec[0]
            for i in range(1, 16):
                s = op(s, vec[i])
            return s

        def count_pass(mid):
            def row(r, acc):
                a = acc
                for j in range(W // 16):
                    v = img[r, pl.ds(j * 16, 16)]
                    a = a + jnp.where(v >= mid, 1.0, 0.0)
                return a
            acc = lax.fori_loop(0, H, row, jnp.zeros((16,), jnp.float32))
            return lane_fold(acc, jnp.add)

        def one_image(i, pooled_vec):
            pltpu.sync_copy(x_hbm.at[b, c0 + cg * 16 + i], img)

            def row_mm(r, carry):
                mx, mn = carry
                for j in range(W // 16):
                    v = img[r, pl.ds(j * 16, 16)]
                    mx = jnp.maximum(mx, v)
                    mn = jnp.minimum(mn, v)
                return mx, mn
            big = jnp.full((16,), -3.0e38, jnp.float32)
            mx, mn = lax.fori_loop(0, H, row_mm, (big, -big))
            maxv = lane_fold(mx, jnp.maximum)
            minv = lane_fold(mn, jnp.minimum)

            def step(_, carry):
                lo, hi = carry
                mid = 0.5 * (lo + hi)
                ok = count_pass(mid) >= kf
                return jnp.where(ok, mid, lo), jnp.where(ok, hi, mid)

            lo, _ = lax.fori_loop(0, _ITERS, step, (minv, maxv))
            t = lo

            def row_fin(r, carry):
                ac, asum = carry
                for j in range(W // 16):
                    v = img[r, pl.ds(j * 16, 16)]
                    ge = v >= t
                    ac = ac + jnp.where(ge, 1.0, 0.0)
                    asum = asum + jnp.where(ge, v, 0.0)
                return ac, asum
            z = jnp.zeros((16,), jnp.float32)
            ac, asum = lax.fori_loop(0, H, row_fin, (z, z))
            cnt_ge = lane_fold(ac, jnp.add)
            sum_ge = lane_fold(asum, jnp.add)
            t_v = jnp.full((16,), t, jnp.float32)
            max_v = jnp.full((16,), maxv, jnp.float32)
            topk_sum = jnp.full((16,), sum_ge - (cnt_ge - kf) * t, jnp.float32)
            avg = topk_sum / kf

            denom = max_v + 1e-6
            s = jnp.minimum(t_v / denom, max_v / denom)
            logits = (s - ts_vec) / 0.1
            gate_soft = 1.0 / (1.0 + jnp.exp(-logits))
            gate_hard = jnp.where(logits >= 0, 1.0, 0.0)
            gate = (gate_hard - gate_soft) + gate_soft
            pooled = gate * max_v + (1.0 - gate) * avg
            lane = lax.iota(jnp.int32, 16)
            return jnp.where(lane == i, pooled, pooled_vec)

        pooled_vec = lax.fori_loop(0, 16, one_image,
                                   jnp.zeros((16,), jnp.float32))
        pv[...] = pooled_vec
        pltpu.sync_copy(pv, out_hbm.at[b, pl.ds(cg * 16, 16)])

    return body(x, tvec)


def kernel(x, T):
    B, C, H, W = x.shape
    n = H * W
    k = max(1, int(n * 0.1))
    tc_c = C - _SC_C
    assert tc_c % _TC_CBLK == 0 and B * _SC_C == 32 * 16 and W % 16 == 0
    t2 = jnp.reshape(T, (1, 1)).astype(jnp.float32)
    tvec = jnp.full((16,), T, jnp.float32)

    out_tc = _tc_pool(x, t2, k=k, tc_c=tc_c)
    out_sc = _sc_pool(x, tvec, k=k, c0=tc_c)
    return jnp.concatenate([out_tc, out_sc], axis=1)
